# baseline (speedup 1.0000x reference)
import functools

import jax
import jax.numpy as jnp
from jax import lax
from jax.experimental import pallas as pl
from jax.experimental.pallas import tpu as pltpu
from jax.experimental.pallas import tpu_sc as plsc

N_NODES = 10000
N_EDGES = 320000
D = 128

NC = 2
NS = 16
NW = NC * NS
CH = 80
EPW = CH * 128
E_PAD = NW * EPW
CH0 = 112
CH1 = 48
CHM = max(CH0, CH1)
TOTAL_CH = NS * (CH0 + CH1)
N_ACC = 10240
DUMMY = 10200
RPT = N_ACC // NS
HR = N_ACC // 128

_mesh = plsc.VectorSubcoreMesh(core_axis_name="c", subcore_axis_name="s")


@functools.partial(
    pl.kernel,
    out_type=jax.ShapeDtypeStruct((NC, N_ACC, D), jnp.float32),
    mesh=_mesh,
    scratch_types=[
        pltpu.VMEM((CH, 128), jnp.int32),
        pltpu.VMEM((128, D), jnp.float32),
        pltpu.VMEM_SHARED((N_ACC, D), jnp.float32),
        pltpu.SemaphoreType.DMA,
    ],
)
def _deg_kernel(dst_hbm, ones_hbm, z_hbm, out_hbm, di, ones_v, acc, sem):
    c = lax.axis_index("c")
    s = lax.axis_index("s")
    w = c * NS + s
    pltpu.sync_copy(z_hbm.at[pl.ds(s * RPT, RPT)], acc.at[pl.ds(s * RPT, RPT)])
    pltpu.sync_copy(ones_hbm, ones_v)
    pltpu.sync_copy(dst_hbm.at[w], di)
    plsc.subcore_barrier()

    def body(j, carry):
        pltpu.sync_copy(ones_v, acc.at[di.at[j]], add=True)
        return carry

    lax.fori_loop(0, CH, body, 0)
    plsc.subcore_barrier()
    pltpu.sync_copy(acc.at[pl.ds(s * RPT, RPT)], out_hbm.at[c, pl.ds(s * RPT, RPT)])


@functools.partial(
    pl.kernel,
    out_type=jax.ShapeDtypeStruct((NC, N_ACC, D), jnp.float32),
    mesh=_mesh,
    scratch_types=[
        pltpu.VMEM((CH, 128), jnp.int32),
        pltpu.VMEM((CH, 128), jnp.int32),
        pltpu.VMEM((128, D), jnp.float32),
        pltpu.VMEM_SHARED((N_ACC, D), jnp.float32),
        pltpu.SemaphoreType.DMA,
    ],
)
def _agg_kernel(g_hbm, src_hbm, dst_hbm, z_hbm, out_hbm, si, di, rows, acc, sem):
    c = lax.axis_index("c")
    s = lax.axis_index("s")
    w = c * NS + s
    pltpu.sync_copy(z_hbm.at[pl.ds(s * RPT, RPT)], acc.at[pl.ds(s * RPT, RPT)])
    pltpu.sync_copy(src_hbm.at[w], si)
    pltpu.sync_copy(dst_hbm.at[w], di)
    plsc.subcore_barrier()

    def body(j, carry):
        pltpu.async_copy(g_hbm.at[si.at[j]], rows, sem).wait()
        pltpu.sync_copy(rows, acc.at[di.at[j]], add=True)
        return carry

    lax.fori_loop(0, CH, body, 0)
    plsc.subcore_barrier()
    pltpu.sync_copy(acc.at[pl.ds(s * RPT, RPT)], out_hbm.at[c, pl.ds(s * RPT, RPT)])


def _tc_u(degp_ref, o_ref):
    o_ref[...] = lax.rsqrt(degp_ref[0, :N_NODES, :] + degp_ref[1, :N_NODES, :] + 1.0)


def _tc_in(x_ref, w_ref, u_ref, o_ref):
    h = jnp.dot(x_ref[...], w_ref[...], preferred_element_type=jnp.float32)
    o_ref[...] = h * u_ref[...]


def _tc_mid(p_ref, m_ref, u_ref, b_ref, w_ref, o_ref):
    a = p_ref[0, :N_NODES, :] + p_ref[1, :N_NODES, :] + m_ref[...]
    t = jnp.maximum(u_ref[...] * a + b_ref[...], 0.0)
    h = jnp.dot(t, w_ref[...], preferred_element_type=jnp.float32)
    o_ref[...] = h * u_ref[...]


def _tc_out(p_ref, m_ref, u_ref, b_ref, w_ref, bl_ref, o_ref):
    a = p_ref[0, :N_NODES, :] + p_ref[1, :N_NODES, :] + m_ref[...]
    t = jnp.maximum(u_ref[...] * a + b_ref[...], 0.0)
    o_ref[...] = (
        jnp.dot(t, w_ref[...], preferred_element_type=jnp.float32) + bl_ref[...]
    )


def kernel(x, edge_index, W1, b1, W2, b2, Wl, bl):
    f32 = jnp.float32
    src = edge_index[0].astype(jnp.int32)
    dst = edge_index[1].astype(jnp.int32)
    pad = E_PAD - N_EDGES
    src_f = jnp.concatenate([src, jnp.zeros((pad,), jnp.int32)])
    dst_f = jnp.concatenate([dst, jnp.full((pad,), DUMMY, jnp.int32)])
    dst_p = dst_f.reshape(NW, CH, 128)
    src_p = src_f.reshape(NW, CH, 128)

    z_rows = jnp.zeros((N_ACC, D), f32)
    ones_sq = jnp.ones((128, D), f32)

    degp = _deg_kernel(dst_p, ones_sq, z_rows)

    u_col = pl.pallas_call(
        _tc_u, out_shape=jax.ShapeDtypeStruct((N_NODES, D), f32)
    )(degp)

    m1 = pl.pallas_call(_tc_in, out_shape=jax.ShapeDtypeStruct((N_NODES, D), f32))(
        x, W1, u_col
    )
    p1 = _agg_kernel(m1, src_p, dst_p, z_rows)

    m2 = pl.pallas_call(_tc_mid, out_shape=jax.ShapeDtypeStruct((N_NODES, D), f32))(
        p1, m1, u_col, b1.reshape(1, D), W2
    )
    p2 = _agg_kernel(m2, src_p, dst_p, z_rows)

    out = pl.pallas_call(
        _tc_out, out_shape=jax.ShapeDtypeStruct((N_NODES, bl.shape[0]), f32)
    )(p2, m2, u_col, b2.reshape(1, D), Wl, bl.reshape(1, bl.shape[0]))
    return out


# speedup vs baseline: 1.5204x; 1.5204x over previous
import functools

import jax
import jax.numpy as jnp
from jax import lax
from jax.experimental import pallas as pl
from jax.experimental.pallas import tpu as pltpu
from jax.experimental.pallas import tpu_sc as plsc

N_NODES = 10000
N_EDGES = 320000
D = 128

NC = 2
NS = 16
NW = NC * NS
CH = 79
EPW = CH * 128
E_PAD = NW * EPW
CH0 = 112
CH1 = 48
CHM = max(CH0, CH1)
TOTAL_CH = NS * (CH0 + CH1)
N_ACC = 10240
DUMMY = 10200
RPT = N_ACC // NS
HR = N_ACC // 128

_mesh = plsc.VectorSubcoreMesh(core_axis_name="c", subcore_axis_name="s")


@functools.partial(
    pl.kernel,
    out_type=jax.ShapeDtypeStruct((NC, N_ACC, D), jnp.float32),
    mesh=_mesh,
    scratch_types=[
        pltpu.VMEM((CH, 128), jnp.int32),
        pltpu.VMEM((128, D), jnp.float32),
        pltpu.VMEM_SHARED((N_ACC, D), jnp.float32),
        pltpu.SemaphoreType.DMA,
    ],
)
def _deg_kernel(dst_hbm, ones_hbm, z_hbm, out_hbm, di, ones_v, acc, sem):
    c = lax.axis_index("c")
    s = lax.axis_index("s")
    w = c * NS + s
    pltpu.sync_copy(z_hbm.at[pl.ds(s * RPT, RPT)], acc.at[pl.ds(s * RPT, RPT)])
    pltpu.sync_copy(ones_hbm, ones_v)
    pltpu.sync_copy(dst_hbm.at[w], di)
    plsc.subcore_barrier()

    def body(j, carry):
        pltpu.sync_copy(ones_v, acc.at[di.at[j]], add=True)
        return carry

    lax.fori_loop(0, CH, body, 0)
    plsc.subcore_barrier()
    pltpu.sync_copy(acc.at[pl.ds(s * RPT, RPT)], out_hbm.at[c, pl.ds(s * RPT, RPT)])


@functools.partial(
    pl.kernel,
    out_type=jax.ShapeDtypeStruct((NC, N_ACC, D), jnp.float32),
    mesh=_mesh,
    scratch_types=[
        pltpu.VMEM((CH, 128), jnp.int32),
        pltpu.VMEM((CH, 128), jnp.int32),
        pltpu.VMEM((128, D), jnp.float32),
        pltpu.VMEM_SHARED((N_ACC, D), jnp.float32),
        pltpu.SemaphoreType.DMA,
    ],
)
def _agg_kernel(g_hbm, src_hbm, dst_hbm, z_hbm, out_hbm, si, di, rows, acc, sem):
    c = lax.axis_index("c")
    s = lax.axis_index("s")
    w = c * NS + s
    pltpu.sync_copy(z_hbm.at[pl.ds(s * RPT, RPT)], acc.at[pl.ds(s * RPT, RPT)])
    pltpu.sync_copy(src_hbm.at[w], si)
    pltpu.sync_copy(dst_hbm.at[w], di)
    plsc.subcore_barrier()

    def body(j, carry):
        pltpu.async_copy(g_hbm.at[si.at[j]], rows, sem).wait()
        pltpu.sync_copy(rows, acc.at[di.at[j]], add=True)
        return carry

    lax.fori_loop(0, CH, body, 0)
    plsc.subcore_barrier()
    pltpu.sync_copy(acc.at[pl.ds(s * RPT, RPT)], out_hbm.at[c, pl.ds(s * RPT, RPT)])


def _tc_u(degp_ref, o_ref):
    o_ref[...] = lax.rsqrt(degp_ref[0, :N_NODES, :] + degp_ref[1, :N_NODES, :] + 1.0)


def _tc_in(x_ref, w_ref, u_ref, o_ref):
    h = jnp.dot(x_ref[...], w_ref[...], preferred_element_type=jnp.float32)
    o_ref[...] = h * u_ref[...]


def _tc_mid(p_ref, m_ref, u_ref, b_ref, w_ref, o_ref):
    a = p_ref[0, :N_NODES, :] + p_ref[1, :N_NODES, :] + m_ref[...]
    t = jnp.maximum(u_ref[...] * a + b_ref[...], 0.0)
    h = jnp.dot(t, w_ref[...], preferred_element_type=jnp.float32)
    o_ref[...] = h * u_ref[...]


def _tc_out(p_ref, m_ref, u_ref, b_ref, w_ref, bl_ref, o_ref):
    a = p_ref[0, :N_NODES, :] + p_ref[1, :N_NODES, :] + m_ref[...]
    t = jnp.maximum(u_ref[...] * a + b_ref[...], 0.0)
    o_ref[...] = (
        jnp.dot(t, w_ref[...], preferred_element_type=jnp.float32) + bl_ref[...]
    )


def kernel(x, edge_index, W1, b1, W2, b2, Wl, bl):
    f32 = jnp.float32
    src = edge_index[0].astype(jnp.int32)
    dst = edge_index[1].astype(jnp.int32)
    pad = E_PAD - N_EDGES
    src_f = jnp.concatenate([src, jnp.zeros((pad,), jnp.int32)])
    dst_f = jnp.concatenate([dst, jnp.full((pad,), DUMMY, jnp.int32)])
    dst_p = dst_f.reshape(NW, CH, 128)
    src_p = src_f.reshape(NW, CH, 128)

    z_rows = jnp.zeros((N_ACC, D), f32)
    ones_sq = jnp.ones((128, D), f32)

    degp = _deg_kernel(dst_p, ones_sq, z_rows)

    u_col = pl.pallas_call(
        _tc_u, out_shape=jax.ShapeDtypeStruct((N_NODES, D), f32)
    )(degp)

    m1 = pl.pallas_call(_tc_in, out_shape=jax.ShapeDtypeStruct((N_NODES, D), f32))(
        x, W1, u_col
    )
    p1 = _agg_kernel(m1, src_p, dst_p, z_rows)

    m2 = pl.pallas_call(_tc_mid, out_shape=jax.ShapeDtypeStruct((N_NODES, D), f32))(
        p1, m1, u_col, b1.reshape(1, D), W2
    )
    p2 = _agg_kernel(m2, src_p, dst_p, z_rows)

    out = pl.pallas_call(
        _tc_out, out_shape=jax.ShapeDtypeStruct((N_NODES, bl.shape[0]), f32)
    )(p2, m2, u_col, b2.reshape(1, D), Wl, bl.reshape(1, bl.shape[0]))
    return out


# speedup vs baseline: 2.6880x; 1.7679x over previous
import functools

import jax
import jax.numpy as jnp
from jax import lax
from jax.experimental import pallas as pl
from jax.experimental.pallas import tpu as pltpu
from jax.experimental.pallas import tpu_sc as plsc

N_NODES = 10000
N_EDGES = 320000
D = 128

NC = 2
NS = 16
NW = NC * NS
CH = 79
EPW = CH * 128
E_PAD = NW * EPW
CH0 = 112
CH1 = 48
CHM = max(CH0, CH1)
TOTAL_CH = NS * (CH0 + CH1)
N_ACC = 10240
DUMMY = 10200
RPT = N_ACC // NS
HR = N_ACC // 128

_mesh = plsc.VectorSubcoreMesh(core_axis_name="c", subcore_axis_name="s")


@functools.partial(
    pl.kernel,
    out_type=jax.ShapeDtypeStruct((NC, N_ACC, D), jnp.float32),
    mesh=_mesh,
    scratch_types=[
        pltpu.VMEM((CH, 128), jnp.int32),
        pltpu.VMEM((128, D), jnp.float32),
        pltpu.VMEM_SHARED((N_ACC, D), jnp.float32),
        pltpu.SemaphoreType.DMA,
    ],
)
def _deg_kernel(dst_hbm, ones_hbm, z_hbm, out_hbm, di, ones_v, acc, sem):
    c = lax.axis_index("c")
    s = lax.axis_index("s")
    w = c * NS + s
    pltpu.sync_copy(z_hbm.at[pl.ds(s * RPT, RPT)], acc.at[pl.ds(s * RPT, RPT)])
    pltpu.sync_copy(ones_hbm, ones_v)
    pltpu.sync_copy(dst_hbm.at[w], di)
    plsc.subcore_barrier()

    def body(j, carry):
        pltpu.sync_copy(ones_v, acc.at[di.at[j]], add=True)
        return carry

    lax.fori_loop(0, CH, body, 0)
    plsc.subcore_barrier()
    pltpu.sync_copy(acc.at[pl.ds(s * RPT, RPT)], out_hbm.at[c, pl.ds(s * RPT, RPT)])


@functools.partial(
    pl.kernel,
    out_type=jax.ShapeDtypeStruct((NC, N_ACC, D), jnp.float32),
    mesh=_mesh,
    scratch_types=[
        pltpu.VMEM((CH, 128), jnp.int32),
        pltpu.VMEM((CH, 128), jnp.int32),
        pltpu.VMEM((128, D), jnp.float32),
        pltpu.VMEM_SHARED((N_ACC, D), jnp.float32),
        pltpu.SemaphoreType.DMA,
    ],
)
def _agg_kernel(g_hbm, src_hbm, dst_hbm, z_hbm, out_hbm, si, di, rows, acc, sem):
    c = lax.axis_index("c")
    s = lax.axis_index("s")
    w = c * NS + s
    pltpu.sync_copy(z_hbm.at[pl.ds(s * RPT, RPT)], acc.at[pl.ds(s * RPT, RPT)])
    pltpu.sync_copy(src_hbm.at[w], si)
    pltpu.sync_copy(dst_hbm.at[w], di)
    plsc.subcore_barrier()

    def body(j, carry):
        pltpu.async_copy(g_hbm.at[si.at[j]], rows, sem).wait()
        pltpu.sync_copy(rows, acc.at[di.at[j]], add=True)
        return carry

    lax.fori_loop(0, CH, body, 0)
    plsc.subcore_barrier()
    pltpu.sync_copy(acc.at[pl.ds(s * RPT, RPT)], out_hbm.at[c, pl.ds(s * RPT, RPT)])


def _tc_u(degp_ref, o_ref):
    o_ref[...] = lax.rsqrt(degp_ref[0, :N_NODES, :] + degp_ref[1, :N_NODES, :] + 1.0)


def _tc_in(x_ref, w_ref, u_ref, o_ref):
    h = jnp.dot(x_ref[...], w_ref[...], preferred_element_type=jnp.float32)
    o_ref[...] = h * u_ref[...]


def _tc_mid(p_ref, m_ref, u_ref, b_ref, w_ref, o_ref):
    a = p_ref[0, :N_NODES, :] + p_ref[1, :N_NODES, :] + m_ref[...]
    t = jnp.maximum(u_ref[...] * a + b_ref[...], 0.0)
    h = jnp.dot(t, w_ref[...], preferred_element_type=jnp.float32)
    o_ref[...] = h * u_ref[...]


def _tc_out(p_ref, m_ref, u_ref, b_ref, w_ref, bl_ref, o_ref):
    a = p_ref[0, :N_NODES, :] + p_ref[1, :N_NODES, :] + m_ref[...]
    t = jnp.maximum(u_ref[...] * a + b_ref[...], 0.0)
    o_ref[...] = (
        jnp.dot(t, w_ref[...], preferred_element_type=jnp.float32) + bl_ref[...]
    )


def kernel(x, edge_index, W1, b1, W2, b2, Wl, bl):
    f32 = jnp.float32
    src = edge_index[0].astype(jnp.int32)
    dst = edge_index[1].astype(jnp.int32)
    pad = E_PAD - N_EDGES
    pad_src = jnp.arange(pad, dtype=jnp.int32) % N_NODES
    pad_dst = N_NODES + jnp.arange(pad, dtype=jnp.int32) % (N_ACC - N_NODES)
    src_f = jnp.concatenate([src, pad_src])
    dst_f = jnp.concatenate([dst, pad_dst])
    dst_p = dst_f.reshape(NW, CH, 128)
    src_p = src_f.reshape(NW, CH, 128)

    z_rows = jnp.zeros((N_ACC, D), f32)
    ones_sq = jnp.ones((128, D), f32)

    degp = _deg_kernel(dst_p, ones_sq, z_rows)

    u_col = pl.pallas_call(
        _tc_u, out_shape=jax.ShapeDtypeStruct((N_NODES, D), f32)
    )(degp)

    m1 = pl.pallas_call(_tc_in, out_shape=jax.ShapeDtypeStruct((N_NODES, D), f32))(
        x, W1, u_col
    )
    p1 = _agg_kernel(m1, src_p, dst_p, z_rows)

    m2 = pl.pallas_call(_tc_mid, out_shape=jax.ShapeDtypeStruct((N_NODES, D), f32))(
        p1, m1, u_col, b1.reshape(1, D), W2
    )
    p2 = _agg_kernel(m2, src_p, dst_p, z_rows)

    out = pl.pallas_call(
        _tc_out, out_shape=jax.ShapeDtypeStruct((N_NODES, bl.shape[0]), f32)
    )(p2, m2, u_col, b2.reshape(1, D), Wl, bl.reshape(1, bl.shape[0]))
    return out
